# initial kernel scaffold (unmeasured)
import jax
import jax.numpy as jnp
from jax import lax
from jax.experimental import pallas as pl
from jax.experimental.pallas import tpu as pltpu

N_DEV = 4
M_BLK = 512


def kernel(x, w_mat, scale_x, scale_w):
    m, k_shard = x.shape
    _, n = w_mat.shape

    def body(x_ref, w_ref, sx_ref, sw_ref, out_ref,
             xs_ref, ws_ref, send_x, recv_x, send_w, recv_w):
        my = lax.axis_index("i")
        left = (my - 1) % N_DEV
        right = (my + 1) % N_DEV

        barrier = pltpu.get_barrier_semaphore()
        for nbr in (left, right):
            pl.semaphore_signal(barrier, inc=1, device_id=(nbr,),
                                device_id_type=pl.DeviceIdType.MESH)
        pl.semaphore_wait(barrier, 2)

        def accum(x_src, w_src, first):
            for mb in range(0, m, M_BLK):
                xb = x_src[pl.ds(mb, M_BLK), :].astype(jnp.bfloat16)
                wb = w_src[:, :].astype(jnp.bfloat16)
                part = jnp.dot(xb, wb, preferred_element_type=jnp.float32)
                if first:
                    out_ref[pl.ds(mb, M_BLK), :] = part
                else:
                    out_ref[pl.ds(mb, M_BLK), :] += part

        for h in range(N_DEV - 1):
            x_src = x_ref if h == 0 else xs_ref.at[h - 1]
            w_src = w_ref if h == 0 else ws_ref.at[h - 1]
            rx = pltpu.make_async_remote_copy(
                src_ref=x_src, dst_ref=xs_ref.at[h],
                send_sem=send_x.at[h], recv_sem=recv_x.at[h],
                device_id=(right,), device_id_type=pl.DeviceIdType.MESH)
            rw = pltpu.make_async_remote_copy(
                src_ref=w_src, dst_ref=ws_ref.at[h],
                send_sem=send_w.at[h], recv_sem=recv_w.at[h],
                device_id=(right,), device_id_type=pl.DeviceIdType.MESH)
            rx.start()
            rw.start()
            accum(x_ref if h == 0 else xs_ref[h - 1],
                  w_ref if h == 0 else ws_ref[h - 1], first=(h == 0))
            rx.wait()
            rw.wait()

        accum(xs_ref[N_DEV - 2], ws_ref[N_DEV - 2], first=False)
        s = sx_ref[0] * sw_ref[0]
        for mb in range(0, m, M_BLK):
            out_ref[pl.ds(mb, M_BLK), :] *= s

    return pl.pallas_call(
        body,
        out_shape=jax.ShapeDtypeStruct((m, n), jnp.float32),
        in_specs=[
            pl.BlockSpec(memory_space=pltpu.VMEM),
            pl.BlockSpec(memory_space=pltpu.VMEM),
            pl.BlockSpec(memory_space=pltpu.SMEM),
            pl.BlockSpec(memory_space=pltpu.SMEM),
        ],
        out_specs=pl.BlockSpec(memory_space=pltpu.VMEM),
        scratch_shapes=[
            pltpu.VMEM((N_DEV - 1, m, k_shard), x.dtype),
            pltpu.VMEM((N_DEV - 1, k_shard, n), w_mat.dtype),
            pltpu.SemaphoreType.DMA((N_DEV - 1,)),
            pltpu.SemaphoreType.DMA((N_DEV - 1,)),
            pltpu.SemaphoreType.DMA((N_DEV - 1,)),
            pltpu.SemaphoreType.DMA((N_DEV - 1,)),
        ],
        compiler_params=pltpu.CompilerParams(
            collective_id=0,
            vmem_limit_bytes=100 * 1024 * 1024,
        ),
    )(x, w_mat, scale_x, scale_w)


# baseline (device time: 272400 ns/iter reference)
import jax
import jax.numpy as jnp
from jax import lax
from jax.experimental import pallas as pl
from jax.experimental.pallas import tpu as pltpu

N_DEV = 4
M_BLK = 256


def kernel(x, w_mat, scale_x, scale_w):
    m, k_shard = x.shape
    _, n = w_mat.shape

    x8 = x.astype(jnp.float8_e4m3fn)
    w8 = w_mat.astype(jnp.float8_e5m2)

    def body(x_ref, w_ref, sx_ref, sw_ref, out_ref,
             xs_ref, ws_ref, send_x, recv_x, send_w, recv_w):
        my = lax.axis_index("i")
        left = (my - 1) % N_DEV
        right = (my + 1) % N_DEV

        barrier = pltpu.get_barrier_semaphore()
        for nbr in (left, right):
            pl.semaphore_signal(barrier, inc=1, device_id=(nbr,),
                                device_id_type=pl.DeviceIdType.MESH)
        pl.semaphore_wait(barrier, 2)

        def accum(x_src, w_src, first):
            for mb in range(0, m, M_BLK):
                part = lax.dot_general(
                    x_src[pl.ds(mb, M_BLK), :], w_src[:, :],
                    (((1,), (0,)), ((), ())),
                    preferred_element_type=jnp.float32)
                if first:
                    out_ref[pl.ds(mb, M_BLK), :] = part
                else:
                    out_ref[pl.ds(mb, M_BLK), :] += part

        for h in range(N_DEV - 1):
            x_src = x_ref if h == 0 else xs_ref.at[h - 1]
            w_src = w_ref if h == 0 else ws_ref.at[h - 1]
            rx = pltpu.make_async_remote_copy(
                src_ref=x_src, dst_ref=xs_ref.at[h],
                send_sem=send_x.at[h], recv_sem=recv_x.at[h],
                device_id=(right,), device_id_type=pl.DeviceIdType.MESH)
            rw = pltpu.make_async_remote_copy(
                src_ref=w_src, dst_ref=ws_ref.at[h],
                send_sem=send_w.at[h], recv_sem=recv_w.at[h],
                device_id=(right,), device_id_type=pl.DeviceIdType.MESH)
            rx.start()
            rw.start()
            accum(x_src, w_src, first=(h == 0))
            rx.wait()
            rw.wait()

        accum(xs_ref.at[N_DEV - 2], ws_ref.at[N_DEV - 2], first=False)
        s = sx_ref[0] * sw_ref[0]
        for mb in range(0, m, M_BLK):
            out_ref[pl.ds(mb, M_BLK), :] *= s

    return pl.pallas_call(
        body,
        out_shape=jax.ShapeDtypeStruct((m, n), jnp.float32),
        in_specs=[
            pl.BlockSpec(memory_space=pltpu.VMEM),
            pl.BlockSpec(memory_space=pltpu.VMEM),
            pl.BlockSpec(memory_space=pltpu.SMEM),
            pl.BlockSpec(memory_space=pltpu.SMEM),
        ],
        out_specs=pl.BlockSpec(memory_space=pltpu.VMEM),
        scratch_shapes=[
            pltpu.VMEM((N_DEV - 1, m, k_shard), jnp.float8_e4m3fn),
            pltpu.VMEM((N_DEV - 1, k_shard, n), jnp.float8_e5m2),
            pltpu.SemaphoreType.DMA((N_DEV - 1,)),
            pltpu.SemaphoreType.DMA((N_DEV - 1,)),
            pltpu.SemaphoreType.DMA((N_DEV - 1,)),
            pltpu.SemaphoreType.DMA((N_DEV - 1,)),
        ],
        compiler_params=pltpu.CompilerParams(
            collective_id=0,
            vmem_limit_bytes=100 * 1024 * 1024,
        ),
    )(x8, w8, scale_x, scale_w)


# device time: 169112 ns/iter; 1.6108x vs baseline; 1.6108x over previous
import jax
import jax.numpy as jnp
from jax import lax
from jax.experimental import pallas as pl
from jax.experimental.pallas import tpu as pltpu

N_DEV = 4
M_BLK = 256


def kernel(x, w_mat, scale_x, scale_w):
    m, k = x.shape
    _, n = w_mat.shape
    m2, k2 = m // 2, k // 2

    x8 = x.astype(jnp.float8_e4m3fn)
    w8 = w_mat.astype(jnp.float8_e5m2)

    def body(x_ref, w_ref, sx_ref, sw_ref, out_ref,
             xL, xR, xD, wL, wR, wD, send, recv):
        my = lax.axis_index("i")
        left = (my - 1) % N_DEV
        right = (my + 1) % N_DEV

        barrier = pltpu.get_barrier_semaphore()
        for nbr in (left, right):
            pl.semaphore_signal(barrier, inc=1, device_id=(nbr,),
                                device_id_type=pl.DeviceIdType.MESH)
        pl.semaphore_wait(barrier, 2)

        def rdma(i, src, dst, dev):
            return pltpu.make_async_remote_copy(
                src_ref=src, dst_ref=dst,
                send_sem=send.at[i], recv_sem=recv.at[i],
                device_id=(dev,), device_id_type=pl.DeviceIdType.MESH)

        def accum(x_src, w_src, mode):
            for mb in range(0, m, M_BLK):
                part = lax.dot_general(
                    x_src[pl.ds(mb, M_BLK), :], w_src[:, :],
                    (((1,), (0,)), ((), ())),
                    preferred_element_type=jnp.float32)
                if mode == 0:
                    out_ref[pl.ds(mb, M_BLK), :] = part
                elif mode == 1:
                    out_ref[pl.ds(mb, M_BLK), :] += part
                else:
                    s = sx_ref[0] * sw_ref[0]
                    out_ref[pl.ds(mb, M_BLK), :] = (
                        out_ref[pl.ds(mb, M_BLK), :] + part) * s

        a_ops = [
            rdma(0, x_ref, xL, right),
            rdma(1, w_ref, wL, right),
            rdma(2, x_ref, xR, left),
            rdma(3, w_ref, wR, left),
        ]
        for op in a_ops:
            op.start()
        accum(x_ref, w_ref, mode=0)
        for op in a_ops:
            op.wait()

        b_ops = [
            rdma(4, xL.at[pl.ds(0, m2)], xD.at[pl.ds(0, m2)], right),
            rdma(5, wL.at[pl.ds(0, k2)], wD.at[pl.ds(0, k2)], right),
            rdma(6, xR.at[pl.ds(m2, m2)], xD.at[pl.ds(m2, m2)], left),
            rdma(7, wR.at[pl.ds(k2, k2)], wD.at[pl.ds(k2, k2)], left),
        ]
        for op in b_ops:
            op.start()
        accum(xL, wL, mode=1)
        accum(xR, wR, mode=1)
        for op in b_ops:
            op.wait()

        accum(xD, wD, mode=2)

    return pl.pallas_call(
        body,
        out_shape=jax.ShapeDtypeStruct((m, n), jnp.float32),
        in_specs=[
            pl.BlockSpec(memory_space=pltpu.VMEM),
            pl.BlockSpec(memory_space=pltpu.VMEM),
            pl.BlockSpec(memory_space=pltpu.SMEM),
            pl.BlockSpec(memory_space=pltpu.SMEM),
        ],
        out_specs=pl.BlockSpec(memory_space=pltpu.VMEM),
        scratch_shapes=[
            pltpu.VMEM((m, k), jnp.float8_e4m3fn),
            pltpu.VMEM((m, k), jnp.float8_e4m3fn),
            pltpu.VMEM((m, k), jnp.float8_e4m3fn),
            pltpu.VMEM((k, n), jnp.float8_e5m2),
            pltpu.VMEM((k, n), jnp.float8_e5m2),
            pltpu.VMEM((k, n), jnp.float8_e5m2),
            pltpu.SemaphoreType.DMA((8,)),
            pltpu.SemaphoreType.DMA((8,)),
        ],
        compiler_params=pltpu.CompilerParams(
            collective_id=0,
            vmem_limit_bytes=100 * 1024 * 1024,
        ),
    )(x8, w8, scale_x, scale_w)
